# baseline (speedup 1.0000x reference)
import functools

import jax
import jax.numpy as jnp
from jax import lax
from jax.experimental import pallas as pl
from jax.experimental.pallas import tpu as pltpu
from jax.experimental.pallas import tpu_sc as plsc

B, S, D = 4, 2048, 1024
T, K = 16, 4
TD = D // T
CTX = TD * (1 + K)
HID = TD * 2

BLK = 512
SB = S // BLK

_HIGH = jax.lax.Precision.HIGHEST
_INV_SQRT2 = 0.7071067811865476


def _layernorm(xb, g, bta):
    mu = jnp.mean(xb, axis=-1, keepdims=True)
    xc = xb - mu
    var = jnp.mean(xc * xc, axis=-1, keepdims=True)
    return xc * jax.lax.rsqrt(var + 1e-5) * g + bta


def _stats_body(x_ref, g_ref, b_ref, Wq_ref, bq_ref, Wk_ref, bk_ref,
                out_ref, acc_scr):
    sb = pl.program_id(1)
    xn = _layernorm(x_ref[0], g_ref[...], b_ref[...])
    part = jnp.sum(xn, axis=0, keepdims=True)

    @pl.when(sb == 0)
    def _():
        acc_scr[...] = part

    @pl.when(sb != 0)
    def _():
        acc_scr[...] += part

    @pl.when(sb == SB - 1)
    def _():
        row = acc_scr[...]
        tr = jnp.concatenate(
            [row[:, TD * t:TD * (t + 1)] for t in range(T)], axis=0
        ) * (1.0 / S)

        def proj_norm(W_ref, bias_ref):
            v = jax.lax.dot_general(tr, W_ref[...], (((1,), (0,)), ((), ())),
                                    precision=jax.lax.Precision.DEFAULT,
                                    preferred_element_type=jnp.float32)
            v = v + bias_ref[...]
            n = jnp.sqrt(jnp.sum(v * v, axis=-1, keepdims=True))
            return v / jnp.maximum(n, 1e-12)

        q = proj_norm(Wq_ref, bq_ref)
        k = proj_norm(Wk_ref, bk_ref)
        scores = jax.lax.dot_general(q, k, (((1,), (1,)), ((), ())),
                                     precision=jax.lax.Precision.DEFAULT,
                                     preferred_element_type=jnp.float32)
        row_i = jax.lax.broadcasted_iota(jnp.int32, (T, T), 0)
        col_i = jax.lax.broadcasted_iota(jnp.int32, (T, T), 1)
        out_ref[...] = jnp.where(row_i == col_i, -1e9, scores)


_SC_NC, _SC_NS = 2, 16
_ROWS = B * T
_ROWS_PER_W = _ROWS // (_SC_NC * _SC_NS)


_HALF_ROWS = _ROWS // 2


def _topk_sc_body(scores_hbm, out_hbm, row_v, idx_v, *, base_row):
    wid = lax.axis_index("s") * _SC_NC + lax.axis_index("c")
    r = base_row + wid
    pltpu.sync_copy(scores_hbm.at[pl.ds(r * T, T)], row_v)
    vals = lax.iota(jnp.int32, T)
    _, sv = plsc.sort_key_val(row_v[...], vals, descending=True)
    idx_v[...] = sv
    pltpu.sync_copy(idx_v, out_hbm.at[pl.ds(wid * T, T)])


def _topk_sc_half(scores_flat, base_row):
    mesh = plsc.VectorSubcoreMesh(core_axis_name="c", subcore_axis_name="s")
    fn = functools.partial(
        pl.kernel,
        mesh=mesh,
        out_type=jax.ShapeDtypeStruct((_HALF_ROWS * T,), jnp.int32),
        scratch_types=[
            pltpu.VMEM((T,), jnp.float32),
            pltpu.VMEM((T,), jnp.int32),
        ],
        compiler_params=pltpu.CompilerParams(needs_layout_passes=False),
    )(functools.partial(_topk_sc_body, base_row=base_row))
    return fn(scores_flat)


def _topk_sc(scores_flat):
    lo = _topk_sc_half(scores_flat, 0)
    hi = _topk_sc_half(scores_flat, _HALF_ROWS)
    return jnp.concatenate([lo, hi])


def _main_body(routes_ref, x_ref, g_ref, bt_ref, W1_ref, b1_ref, W2_ref,
               b2_ref, out_ref, tiles_scr):
    b = pl.program_id(0)
    xb = x_ref[0]
    xn = _layernorm(xb, g_ref[...], bt_ref[...]).astype(jnp.bfloat16)
    for t in range(T):
        tiles_scr[t] = xn[:, TD * t:TD * (t + 1)]
    W1 = W1_ref[...]
    W2 = W2_ref[...]
    b1v = b1_ref[...]
    b2v = b2_ref[...]
    combs = []
    for t in range(T):
        base = (b * T + t) * T
        parts = [tiles_scr[t]]
        for kk in range(K):
            parts.append(tiles_scr[routes_ref[base + kk]])
        combs.append(jnp.concatenate(parts, axis=1))
    comb = jnp.concatenate(combs, axis=0)
    h = jax.lax.dot_general(comb, W1, (((1,), (0,)), ((), ())),
                            preferred_element_type=jnp.float32) + b1v
    h = h * (0.5 * jax.lax.erf(h * _INV_SQRT2) + 0.5)
    o = jax.lax.dot_general(h.astype(jnp.bfloat16), W2,
                            (((1,), (0,)), ((), ())),
                            preferred_element_type=jnp.float32) + b2v
    for t in range(0, T, 2):
        out_ref[0, :, TD * t:TD * (t + 2)] = (
            xb[:, TD * t:TD * (t + 2)]
            + jnp.concatenate(
                [o[BLK * t:BLK * (t + 1)], o[BLK * (t + 1):BLK * (t + 2)]],
                axis=1))


@jax.jit
def kernel(x, ln_g, ln_b, Wq, bq, Wk, bk, W1, b1, W2, b2):
    g2 = ln_g.reshape(1, D)
    bt2 = ln_b.reshape(1, D)

    scores = pl.pallas_call(
        _stats_body,
        grid=(B, SB),
        in_specs=[
            pl.BlockSpec((1, BLK, D), lambda b_, s_: (b_, s_, 0)),
            pl.BlockSpec((1, D), lambda b_, s_: (0, 0)),
            pl.BlockSpec((1, D), lambda b_, s_: (0, 0)),
            pl.BlockSpec((TD, TD), lambda b_, s_: (0, 0)),
            pl.BlockSpec((1, TD), lambda b_, s_: (0, 0)),
            pl.BlockSpec((TD, TD), lambda b_, s_: (0, 0)),
            pl.BlockSpec((1, TD), lambda b_, s_: (0, 0)),
        ],
        out_specs=pl.BlockSpec((T, T), lambda b_, s_: (b_, 0)),
        out_shape=jax.ShapeDtypeStruct((_ROWS, T), jnp.float32),
        scratch_shapes=[pltpu.VMEM((1, D), jnp.float32)],
        compiler_params=pltpu.CompilerParams(
            dimension_semantics=("arbitrary", "arbitrary")),
    )(x, g2, bt2, Wq, bq.reshape(1, TD), Wk, bk.reshape(1, TD))

    ranked = _topk_sc(scores.reshape(-1))

    out = pl.pallas_call(
        _main_body,
        grid_spec=pltpu.PrefetchScalarGridSpec(
            num_scalar_prefetch=1,
            grid=(B, SB),
            in_specs=[
                pl.BlockSpec((1, BLK, D), lambda b_, s_, *_: (b_, s_, 0)),
                pl.BlockSpec((1, D), lambda b_, s_, *_: (0, 0)),
                pl.BlockSpec((1, D), lambda b_, s_, *_: (0, 0)),
                pl.BlockSpec((CTX, HID), lambda b_, s_, *_: (0, 0)),
                pl.BlockSpec((1, HID), lambda b_, s_, *_: (0, 0)),
                pl.BlockSpec((HID, TD), lambda b_, s_, *_: (0, 0)),
                pl.BlockSpec((1, TD), lambda b_, s_, *_: (0, 0)),
            ],
            out_specs=pl.BlockSpec((1, BLK, D), lambda b_, s_, *_: (b_, s_, 0)),
            scratch_shapes=[pltpu.VMEM((T, BLK, TD), jnp.bfloat16)],
        ),
        out_shape=jax.ShapeDtypeStruct((B, S, D), jnp.float32),
        compiler_params=pltpu.CompilerParams(
            dimension_semantics=("arbitrary", "arbitrary")),
    )(ranked, x, g2, bt2, W1.astype(jnp.bfloat16),
      b1.reshape(1, HID), W2.astype(jnp.bfloat16), b2.reshape(1, TD))

    return out


# speedup vs baseline: 1.1243x; 1.1243x over previous
import functools

import jax
import jax.numpy as jnp
from jax import lax
from jax.experimental import pallas as pl
from jax.experimental.pallas import tpu as pltpu
from jax.experimental.pallas import tpu_sc as plsc

B, S, D = 4, 2048, 1024
T, K = 16, 4
TD = D // T
CTX = TD * (1 + K)
HID = TD * 2

BLK = 1024
SB = S // BLK

_HIGH = jax.lax.Precision.HIGHEST
_INV_SQRT2 = 0.7071067811865476


def _layernorm(xb, g, bta):
    mu = jnp.mean(xb, axis=-1, keepdims=True)
    xc = xb - mu
    var = jnp.mean(xc * xc, axis=-1, keepdims=True)
    return xc * jax.lax.rsqrt(var + 1e-5) * g + bta


def _stats_body(x_ref, g_ref, b_ref, Wq_ref, bq_ref, Wk_ref, bk_ref,
                out_ref, acc_scr):
    sb = pl.program_id(1)
    xn = _layernorm(x_ref[0], g_ref[...], b_ref[...])
    part = jnp.sum(xn, axis=0, keepdims=True)

    @pl.when(sb == 0)
    def _():
        acc_scr[...] = part

    @pl.when(sb != 0)
    def _():
        acc_scr[...] += part

    @pl.when(sb == SB - 1)
    def _():
        row = acc_scr[...]
        tr = jnp.concatenate(
            [row[:, TD * t:TD * (t + 1)] for t in range(T)], axis=0
        ) * (1.0 / S)

        def proj_norm(W_ref, bias_ref):
            v = jax.lax.dot_general(tr, W_ref[...], (((1,), (0,)), ((), ())),
                                    precision=jax.lax.Precision.DEFAULT,
                                    preferred_element_type=jnp.float32)
            v = v + bias_ref[...]
            n = jnp.sqrt(jnp.sum(v * v, axis=-1, keepdims=True))
            return v / jnp.maximum(n, 1e-12)

        q = proj_norm(Wq_ref, bq_ref)
        k = proj_norm(Wk_ref, bk_ref)
        scores = jax.lax.dot_general(q, k, (((1,), (1,)), ((), ())),
                                     precision=jax.lax.Precision.DEFAULT,
                                     preferred_element_type=jnp.float32)
        row_i = jax.lax.broadcasted_iota(jnp.int32, (T, T), 0)
        col_i = jax.lax.broadcasted_iota(jnp.int32, (T, T), 1)
        out_ref[...] = jnp.where(row_i == col_i, -1e9, scores)


_SC_NC, _SC_NS = 2, 16
_ROWS = B * T
_ROWS_PER_W = _ROWS // (_SC_NC * _SC_NS)


def _topk_sc_body(scores_hbm, out_hbm, rows_v, idxs_v):
    wid = lax.axis_index("s") * _SC_NC + lax.axis_index("c")
    base = wid * (_ROWS_PER_W * T)
    pltpu.sync_copy(scores_hbm.at[pl.ds(base, _ROWS_PER_W * T)], rows_v)
    vals = lax.iota(jnp.int32, T)

    def _row(i, carry):
        _, sv = plsc.sort_key_val(rows_v[pl.ds(i * T, T)], vals,
                                  descending=True)
        idxs_v[pl.ds(i * T, T)] = sv
        return carry

    lax.fori_loop(0, _ROWS_PER_W, _row, 0)
    pltpu.sync_copy(idxs_v, out_hbm.at[pl.ds(base, _ROWS_PER_W * T)])


def _topk_sc(scores_flat):
    mesh = plsc.VectorSubcoreMesh(core_axis_name="c", subcore_axis_name="s")
    fn = functools.partial(
        pl.kernel,
        mesh=mesh,
        out_type=jax.ShapeDtypeStruct((_ROWS * T,), jnp.int32),
        scratch_types=[
            pltpu.VMEM((_ROWS_PER_W * T,), jnp.float32),
            pltpu.VMEM((_ROWS_PER_W * T,), jnp.int32),
        ],
        compiler_params=pltpu.CompilerParams(needs_layout_passes=False),
    )(_topk_sc_body)
    return fn(scores_flat)


def _main_body(routes_ref, x_ref, g_ref, bt_ref, W1_ref, b1_ref, W2_ref,
               b2_ref, out_ref, tiles_scr):
    b = pl.program_id(0)
    xb = x_ref[0]
    xn = _layernorm(xb, g_ref[...], bt_ref[...]).astype(jnp.bfloat16)
    for t in range(T):
        tiles_scr[t] = xn[:, TD * t:TD * (t + 1)]
    W1 = W1_ref[...]
    W2 = W2_ref[...]
    b1v = b1_ref[...]
    b2v = b2_ref[...]
    combs = []
    for t in range(T):
        base = (b * T + t) * T
        parts = [tiles_scr[t]]
        for kk in range(K):
            parts.append(tiles_scr[routes_ref[base + kk]])
        combs.append(jnp.concatenate(parts, axis=1))
    comb = jnp.concatenate(combs, axis=0)
    h = jax.lax.dot_general(comb, W1, (((1,), (0,)), ((), ())),
                            preferred_element_type=jnp.float32) + b1v
    h = h * (0.5 * jax.lax.erf(h * _INV_SQRT2) + 0.5)
    o = jax.lax.dot_general(h.astype(jnp.bfloat16), W2,
                            (((1,), (0,)), ((), ())),
                            preferred_element_type=jnp.float32) + b2v
    for t in range(0, T, 2):
        out_ref[0, :, TD * t:TD * (t + 2)] = (
            xb[:, TD * t:TD * (t + 2)]
            + jnp.concatenate(
                [o[BLK * t:BLK * (t + 1)], o[BLK * (t + 1):BLK * (t + 2)]],
                axis=1))


@jax.jit
def kernel(x, ln_g, ln_b, Wq, bq, Wk, bk, W1, b1, W2, b2):
    g2 = ln_g.reshape(1, D)
    bt2 = ln_b.reshape(1, D)

    scores = pl.pallas_call(
        _stats_body,
        grid=(B, SB),
        in_specs=[
            pl.BlockSpec((1, BLK, D), lambda b_, s_: (b_, s_, 0)),
            pl.BlockSpec((1, D), lambda b_, s_: (0, 0)),
            pl.BlockSpec((1, D), lambda b_, s_: (0, 0)),
            pl.BlockSpec((TD, TD), lambda b_, s_: (0, 0)),
            pl.BlockSpec((1, TD), lambda b_, s_: (0, 0)),
            pl.BlockSpec((TD, TD), lambda b_, s_: (0, 0)),
            pl.BlockSpec((1, TD), lambda b_, s_: (0, 0)),
        ],
        out_specs=pl.BlockSpec((T, T), lambda b_, s_: (b_, 0)),
        out_shape=jax.ShapeDtypeStruct((_ROWS, T), jnp.float32),
        scratch_shapes=[pltpu.VMEM((1, D), jnp.float32)],
        compiler_params=pltpu.CompilerParams(
            dimension_semantics=("arbitrary", "arbitrary")),
    )(x, g2, bt2, Wq, bq.reshape(1, TD), Wk, bk.reshape(1, TD))

    ranked = _topk_sc(scores.reshape(-1))

    out = pl.pallas_call(
        _main_body,
        grid_spec=pltpu.PrefetchScalarGridSpec(
            num_scalar_prefetch=1,
            grid=(B, SB),
            in_specs=[
                pl.BlockSpec((1, BLK, D), lambda b_, s_, *_: (b_, s_, 0)),
                pl.BlockSpec((1, D), lambda b_, s_, *_: (0, 0)),
                pl.BlockSpec((1, D), lambda b_, s_, *_: (0, 0)),
                pl.BlockSpec((CTX, HID), lambda b_, s_, *_: (0, 0)),
                pl.BlockSpec((1, HID), lambda b_, s_, *_: (0, 0)),
                pl.BlockSpec((HID, TD), lambda b_, s_, *_: (0, 0)),
                pl.BlockSpec((1, TD), lambda b_, s_, *_: (0, 0)),
            ],
            out_specs=pl.BlockSpec((1, BLK, D), lambda b_, s_, *_: (b_, s_, 0)),
            scratch_shapes=[pltpu.VMEM((T, BLK, TD), jnp.bfloat16)],
        ),
        out_shape=jax.ShapeDtypeStruct((B, S, D), jnp.float32),
        compiler_params=pltpu.CompilerParams(
            dimension_semantics=("arbitrary", "arbitrary")),
    )(ranked, x, g2, bt2, W1.astype(jnp.bfloat16),
      b1.reshape(1, HID), W2.astype(jnp.bfloat16), b2.reshape(1, TD))

    return out
